# baseline (speedup 1.0000x reference)
import functools
import math

import jax
import jax.numpy as jnp
from jax.experimental import pallas as pl
from jax.experimental.pallas import tpu as pltpu
from jax.experimental.pallas import tpu_sc as plsc

_H = 4
_D = 32
_HID = 128
_L = 2
_NP = 50000
_NA = 50000
_NT = _NP + _NA


def _mm(x, w, b, act=None, blk=2000):
    n = x.shape[0]
    assert n % blk == 0

    def body(x_ref, w_ref, b_ref, o_ref):
        xv = x_ref[...]
        if act == "gelu_in":
            xv = jax.nn.gelu(xv)
        y = jnp.dot(xv, w_ref[...], preferred_element_type=jnp.float32)
        y = y + b_ref[...]
        if act == "relu":
            y = jnp.maximum(y, 0.0)
        o_ref[...] = y

    return pl.pallas_call(
        body,
        grid=(n // blk,),
        in_specs=[
            pl.BlockSpec((blk, _HID), lambda i: (i, 0)),
            pl.BlockSpec((_HID, _HID), lambda i: (0, 0)),
            pl.BlockSpec((1, _HID), lambda i: (0, 0)),
        ],
        out_specs=pl.BlockSpec((blk, _HID), lambda i: (i, 0)),
        out_shape=jax.ShapeDtypeStruct((n, _HID), jnp.float32),
    )(x, w, b.reshape(1, _HID))


_E = 800000
_NW = 32
_EPW = _E // _NW
_CHUNKS = [(128, 195), (40, 1)]


@functools.partial(jax.jit, static_argnums=())
def _sc_gather3(kt, vt, qt, src_g, dst_g):
    mesh = plsc.VectorSubcoreMesh(core_axis_name="c", subcore_axis_name="s")
    out3 = jax.ShapeDtypeStruct((_E, _HID), jnp.float32)

    @functools.partial(
        pl.kernel,
        mesh=mesh,
        out_type=[out3, out3, out3],
        scratch_types=[
            pltpu.VMEM((128,), jnp.int32),
            pltpu.VMEM((128,), jnp.int32),
            pltpu.VMEM((128, _HID), jnp.float32),
            pltpu.VMEM((128, _HID), jnp.float32),
            pltpu.VMEM((128, _HID), jnp.float32),
            pltpu.SemaphoreType.DMA,
            pltpu.SemaphoreType.DMA,
            pltpu.SemaphoreType.DMA,
        ],
    )
    def k(kt_h, vt_h, qt_h, src_h, dst_h, oke, ove, oqe,
          sidx, didx, kb, vb, qb, s1, s2, s3):
        wid = jax.lax.axis_index("s") * 2 + jax.lax.axis_index("c")
        wbase = wid * _EPW

        def do_chunk(base, c):
            pltpu.sync_copy(src_h.at[pl.ds(base, c)], sidx.at[pl.ds(0, c)])
            pltpu.sync_copy(dst_h.at[pl.ds(base, c)], didx.at[pl.ds(0, c)])
            c1 = pltpu.async_copy(kt_h.at[sidx.at[pl.ds(0, c)]],
                                  kb.at[pl.ds(0, c)], s1)
            c2 = pltpu.async_copy(vt_h.at[sidx.at[pl.ds(0, c)]],
                                  vb.at[pl.ds(0, c)], s2)
            c3 = pltpu.async_copy(qt_h.at[didx.at[pl.ds(0, c)]],
                                  qb.at[pl.ds(0, c)], s3)
            c1.wait()
            c2.wait()
            c3.wait()
            pltpu.sync_copy(kb.at[pl.ds(0, c)], oke.at[pl.ds(base, c)])
            pltpu.sync_copy(vb.at[pl.ds(0, c)], ove.at[pl.ds(base, c)])
            pltpu.sync_copy(qb.at[pl.ds(0, c)], oqe.at[pl.ds(base, c)])

        off = 0
        for c, n in _CHUNKS:
            def step(i, _, c=c, off=off):
                do_chunk(wbase + off + i * c, c)
                return 0
            jax.lax.fori_loop(0, n, step, 0)
            off += c * n

    return k(kt, vt, qt, src_g, dst_g)


def _blockdiag(mats):
    return jax.scipy.linalg.block_diag(*[mats[h] for h in range(_H)])


def kernel(x_paper, x_author, ei_cites, ei_writes, ei_written_by, W_in, b_in,
           Wk, bk, Wq, bq, Wv, bv, Wo, bo, skip, a_rel, m_rel, p_rel,
           bn_gamma, bn_beta):
    xs = [x_paper, x_author]
    eidx = [ei_cites, ei_writes, ei_written_by]
    st_i = [0, 1, 0]
    src_g = jnp.concatenate([eidx[e][0] + e * _NP for e in range(3)])
    dst_g = jnp.concatenate(
        [eidx[0][1], eidx[1][1], eidx[2][1] + _NP])
    perm = jnp.argsort(dst_g)
    src_g = src_g[perm]
    dst_g = dst_g[perm]

    h = [_mm(xs[i], W_in[i], b_in[i], act="relu") for i in range(2)]

    inv_sqrt_d = 1.0 / math.sqrt(float(_D))
    for l in range(_L):
        kt_parts, vt_parts = [], []
        for e in range(3):
            i = st_i[e]
            ablk = _blockdiag(a_rel[l, e] * (p_rel[l, e][:, None, None] * inv_sqrt_d))
            mblk = _blockdiag(m_rel[l, e])
            kt_parts.append(_mm(h[i], Wk[l, i] @ ablk, bk[l, i] @ ablk))
            vt_parts.append(_mm(h[i], Wv[l, i] @ mblk, bv[l, i] @ mblk))
        kt = jnp.concatenate(kt_parts)
        vt = jnp.concatenate(vt_parts)
        qt = jnp.concatenate(
            [_mm(h[i], Wq[l, i], bq[l, i]) for i in range(2)])

        ke, ve, qe = _sc_gather3(kt, vt, qt, src_g, dst_g)
        score = (ke * qe).reshape(-1, _H, _D).sum(-1)
        ex = jnp.exp(score)
        numer = jax.ops.segment_sum(ve * jnp.repeat(ex, _D, axis=1), dst_g,
                                    num_segments=_NT, indices_are_sorted=True)
        den = jax.ops.segment_sum(ex, dst_g, num_segments=_NT,
                                  indices_are_sorted=True)
        agg = numer / (jnp.repeat(den, _D, axis=1) + 1e-16)

        new_h = []
        for i in range(2):
            o = _mm(agg[i * _NP:(i + 1) * _NP], Wo[l, i], bo[l, i], act="gelu_in")
            a = jax.nn.sigmoid(skip[l, i])
            o = a * o + (1.0 - a) * h[i]
            mu = o.mean(0)
            var = o.var(0)
            o = (o - mu) / jnp.sqrt(var + 1e-5) * bn_gamma[l] + bn_beta[l]
            new_h.append(o)
        h = new_h
    return (h[0], h[1])


# speedup vs baseline: 1.0749x; 1.0749x over previous
import functools
import math

import jax
import jax.numpy as jnp
from jax.experimental import pallas as pl
from jax.experimental.pallas import tpu as pltpu
from jax.experimental.pallas import tpu_sc as plsc

_H = 4
_D = 32
_HID = 128
_L = 2
_NP = 50000
_NA = 50000
_NT = _NP + _NA


def _mm(x, w, b, act=None, blk=2000):
    n = x.shape[0]
    assert n % blk == 0

    def body(x_ref, w_ref, b_ref, o_ref):
        xv = x_ref[...]
        if act == "gelu_in":
            xv = jax.nn.gelu(xv)
        y = jnp.dot(xv, w_ref[...], preferred_element_type=jnp.float32)
        y = y + b_ref[...]
        if act == "relu":
            y = jnp.maximum(y, 0.0)
        o_ref[...] = y

    return pl.pallas_call(
        body,
        grid=(n // blk,),
        in_specs=[
            pl.BlockSpec((blk, _HID), lambda i: (i, 0)),
            pl.BlockSpec((_HID, _HID), lambda i: (0, 0)),
            pl.BlockSpec((1, _HID), lambda i: (0, 0)),
        ],
        out_specs=pl.BlockSpec((blk, _HID), lambda i: (i, 0)),
        out_shape=jax.ShapeDtypeStruct((n, _HID), jnp.float32),
    )(x, w, b.reshape(1, _HID))


_E = 800000
_NW = 32
_EPW = _E // _NW
_C = 128
_NCHUNK = 196


@functools.partial(jax.jit, static_argnums=())
def _sc_gather3(kt, vt, qt, src_g, dst_g):
    mesh = plsc.VectorSubcoreMesh(core_axis_name="c", subcore_axis_name="s")
    out3 = jax.ShapeDtypeStruct((_E, _HID), jnp.float32)

    idx_t = pltpu.VMEM((_C,), jnp.int32)
    row_t = pltpu.VMEM((_C, _HID), jnp.float32)
    sem_t = pltpu.SemaphoreType.DMA

    @functools.partial(
        pl.kernel,
        mesh=mesh,
        out_type=[out3, out3, out3],
        scratch_types=[idx_t] * 4 + [row_t] * 6 + [sem_t] * 12,
    )
    def k(kt_h, vt_h, qt_h, src_h, dst_h, oke, ove, oqe,
          sia, dia, sib, dib, ka, va, qa, kb, vb, qb, *sems):
        wid = jax.lax.axis_index("s") * 2 + jax.lax.axis_index("c")
        wbase = wid * _EPW
        ga, gb, wa, wb = sems[0:3], sems[3:6], sems[6:9], sems[9:12]

        def fire(base, si, di, kr, vr, qr, g):
            pltpu.sync_copy(src_h.at[pl.ds(base, _C)], si)
            pltpu.sync_copy(dst_h.at[pl.ds(base, _C)], di)
            return (pltpu.async_copy(kt_h.at[si], kr, g[0]),
                    pltpu.async_copy(vt_h.at[si], vr, g[1]),
                    pltpu.async_copy(qt_h.at[di], qr, g[2]))

        def writeback(base, kr, vr, qr, w):
            return (pltpu.async_copy(kr, oke.at[pl.ds(base, _C)], w[0]),
                    pltpu.async_copy(vr, ove.at[pl.ds(base, _C)], w[1]),
                    pltpu.async_copy(qr, oqe.at[pl.ds(base, _C)], w[2]))

        def step(j, _):
            base_a = wbase + jnp.minimum((2 * j) * _C, _EPW - _C)
            base_b = wbase + jnp.minimum((2 * j + 1) * _C, _EPW - _C)
            ca = fire(base_a, sia, dia, ka, va, qa, ga)
            cb = fire(base_b, sib, dib, kb, vb, qb, gb)
            for c in ca:
                c.wait()
            cwa = writeback(base_a, ka, va, qa, wa)
            for c in cb:
                c.wait()
            cwb = writeback(base_b, kb, vb, qb, wb)
            for c in cwa + cwb:
                c.wait()
            return 0

        jax.lax.fori_loop(0, _NCHUNK // 2, step, 0)

    return k(kt, vt, qt, src_g, dst_g)


def _blockdiag(mats):
    return jax.scipy.linalg.block_diag(*[mats[h] for h in range(_H)])


def kernel(x_paper, x_author, ei_cites, ei_writes, ei_written_by, W_in, b_in,
           Wk, bk, Wq, bq, Wv, bv, Wo, bo, skip, a_rel, m_rel, p_rel,
           bn_gamma, bn_beta):
    xs = [x_paper, x_author]
    eidx = [ei_cites, ei_writes, ei_written_by]
    st_i = [0, 1, 0]
    src_g = jnp.concatenate([eidx[e][0] + e * _NP for e in range(3)])
    dst_g = jnp.concatenate(
        [eidx[0][1], eidx[1][1], eidx[2][1] + _NP])
    perm = jnp.argsort(dst_g)
    src_g = src_g[perm]
    dst_g = dst_g[perm]

    h = [_mm(xs[i], W_in[i], b_in[i], act="relu") for i in range(2)]

    inv_sqrt_d = 1.0 / math.sqrt(float(_D))
    for l in range(_L):
        kt_parts, vt_parts = [], []
        for e in range(3):
            i = st_i[e]
            ablk = _blockdiag(a_rel[l, e] * (p_rel[l, e][:, None, None] * inv_sqrt_d))
            mblk = _blockdiag(m_rel[l, e])
            kt_parts.append(_mm(h[i], Wk[l, i] @ ablk, bk[l, i] @ ablk))
            vt_parts.append(_mm(h[i], Wv[l, i] @ mblk, bv[l, i] @ mblk))
        kt = jnp.concatenate(kt_parts)
        vt = jnp.concatenate(vt_parts)
        qt = jnp.concatenate(
            [_mm(h[i], Wq[l, i], bq[l, i]) for i in range(2)])

        ke, ve, qe = _sc_gather3(kt, vt, qt, src_g, dst_g)
        score = (ke * qe).reshape(-1, _H, _D).sum(-1)
        ex = jnp.exp(score)
        big = jnp.concatenate([ve * jnp.repeat(ex, _D, axis=1), ex], axis=1)
        tot = jax.ops.segment_sum(big, dst_g, num_segments=_NT,
                                  indices_are_sorted=True)
        agg = tot[:, :_HID] / (jnp.repeat(tot[:, _HID:], _D, axis=1) + 1e-16)

        new_h = []
        for i in range(2):
            o = _mm(agg[i * _NP:(i + 1) * _NP], Wo[l, i], bo[l, i], act="gelu_in")
            a = jax.nn.sigmoid(skip[l, i])
            o = a * o + (1.0 - a) * h[i]
            mu = o.mean(0)
            var = o.var(0)
            o = (o - mu) / jnp.sqrt(var + 1e-5) * bn_gamma[l] + bn_beta[l]
            new_h.append(o)
        h = new_h
    return (h[0], h[1])
